# baseline (speedup 1.0000x reference)
import functools

import jax
import jax.numpy as jnp
from jax import lax
from jax.experimental import pallas as pl
from jax.experimental.pallas import tpu as pltpu
from jax.experimental.pallas import tpu_sc as plsc

N = 10000
E = 320000
D = 128
H = 32

NC = 2
NS = 16
NW = NC * NS
CH = 128
K = 4
CHUNKS = 80
E_PAD = NW * CHUNKS * CH
NG = CHUNKS // K
ZR = 640
N_PAD = NS * ZR
ZB = 64
NWIDE = N_PAD // 4
NWR = N // 4


@functools.cache
def _build_edge_scatter():
    mesh = plsc.VectorSubcoreMesh(core_axis_name="c", subcore_axis_name="s")

    @functools.partial(
        pl.kernel,
        mesh=mesh,
        compiler_params=pltpu.CompilerParams(use_tc_tiling_on_sc=False),
        out_type=jax.ShapeDtypeStruct((NC * N_PAD, H), jnp.float32),
        scratch_types=[
            pltpu.VMEM((CHUNKS, CH), jnp.int32),
            pltpu.VMEM((CHUNKS, CH), jnp.int32),
            pltpu.VMEM((2 * K, CH, H), jnp.float32),
            pltpu.VMEM((ZB, H), jnp.float32),
            pltpu.VMEM((ZR, H), jnp.float32),
            pltpu.VMEM_SHARED((N_PAD, H), jnp.float32),
            pltpu.VMEM_SHARED((N_PAD, H), jnp.float32),
            pltpu.SemaphoreType.DMA,
            pltpu.SemaphoreType.DMA,
            pltpu.SemaphoreType.DMA,
            pltpu.SemaphoreType.DMA,
        ],
    )
    def _edge_scatter(y_hbm, src_hbm, dst_hbm, out_hbm,
                      src_v, dst_v, rows_v, zero_v, stage_v, acc_sh, y_sh,
                      gsem0, gsem1, ssem0, ssem1):
        c = lax.axis_index("c")
        s = lax.axis_index("s")
        wid = s * NC + c

        def _zrow(i, carry):
            zero_v[i, pl.ds(0, 16)] = jnp.zeros((16,), jnp.float32)
            zero_v[i, pl.ds(16, 16)] = jnp.zeros((16,), jnp.float32)
            return carry
        lax.fori_loop(0, ZB, _zrow, 0)
        base = s * ZR
        off = 0
        while off < ZR:
            k = min(ZB, ZR - off)
            pltpu.sync_copy(zero_v.at[pl.ds(0, k)],
                            acc_sh.at[pl.ds(base + off, k)])
            off += k
        pltpu.sync_copy(y_hbm.at[pl.ds(base, ZR)], stage_v)
        pltpu.sync_copy(stage_v, y_sh.at[pl.ds(base, ZR)])
        plsc.subcore_barrier()

        pltpu.sync_copy(src_hbm.at[wid], src_v)
        pltpu.sync_copy(dst_hbm.at[wid], dst_v)

        gsems = (gsem0, gsem1)
        ssems = (ssem0, ssem1)
        gds = [None] * CHUNKS
        sds = [None] * CHUNKS

        def _fire_gathers(g):
            bank = g % 2
            for k in range(K):
                j = g * K + k
                gds[j] = pltpu.async_copy(
                    y_sh.at[src_v.at[j]], rows_v.at[bank * K + k],
                    gsems[bank])

        def _fire_scatters(g):
            bank = g % 2
            for k in range(K):
                j = g * K + k
                sds[j] = pltpu.async_copy(
                    rows_v.at[bank * K + k], acc_sh.at[dst_v.at[j]],
                    ssems[bank], add=True)

        _fire_gathers(0)
        for g in range(NG):
            if g + 1 < NG:
                if g >= 1:
                    for k in range(K):
                        sds[(g - 1) * K + k].wait()
                _fire_gathers(g + 1)
            for k in range(K):
                gds[g * K + k].wait()
            _fire_scatters(g)
        for g in (NG - 2, NG - 1):
            for k in range(K):
                sds[g * K + k].wait()
        plsc.subcore_barrier()

        pltpu.sync_copy(acc_sh.at[pl.ds(base, ZR)], stage_v)
        pltpu.sync_copy(stage_v, out_hbm.at[pl.ds(c * N_PAD + base, ZR)])

    return _edge_scatter


def _mm1_body(xw_ref, w_ref, o_ref):
    o_ref[0:NWR, :] = jnp.dot(xw_ref[...], w_ref[...],
                              preferred_element_type=jnp.float32)
    o_ref[NWR:NWIDE, :] = jnp.zeros((NWIDE - NWR, 128), jnp.float32)


def _mid_body(y1_ref, s1_ref, b1a_ref, w2a_ref, b2a_ref, w1b_ref,
              z1_ref, y2_ref):
    s1 = s1_ref[0:NWR, :] + s1_ref[NWIDE:NWIDE + NWR, :]
    t1 = jnp.maximum(y1_ref[0:NWR, :] + s1 + b1a_ref[...], 0.0)
    z1 = jnp.maximum(
        jnp.dot(t1, w2a_ref[...], preferred_element_type=jnp.float32)
        + b2a_ref[...], 0.0)
    z1_ref[...] = z1
    y2_ref[0:NWR, :] = jnp.dot(z1, w1b_ref[...],
                               preferred_element_type=jnp.float32)
    y2_ref[NWR:NWIDE, :] = jnp.zeros((NWIDE - NWR, 128), jnp.float32)


def _out_body(z1_ref, y2_ref, s2_ref, b1b_ref, w2b_ref, b2b_ref, z_ref):
    s2 = s2_ref[0:NWR, :] + s2_ref[NWIDE:NWIDE + NWR, :]
    t2 = jnp.maximum(y2_ref[0:NWR, :] + s2 + b1b_ref[...], 0.0)
    z2 = jnp.maximum(
        jnp.dot(t2, w2b_ref[...], preferred_element_type=jnp.float32)
        + b2b_ref[...], 0.0)
    z1 = z1_ref[...]
    parts = []
    for j in range(4):
        parts.append(z1[:, j * H:(j + 1) * H])
        parts.append(z2[:, j * H:(j + 1) * H])
    z_ref[...] = jnp.concatenate(parts, axis=1)


def _bd(w):
    return jnp.kron(jnp.eye(4, dtype=jnp.float32), w)


def kernel(x, edge_index, batch, W1a, b1a, W2a, b2a, W1b, b1b, W2b, b2b):
    pad = E_PAD - E
    dummy = N + (jnp.arange(pad, dtype=jnp.int32) % (N_PAD - N))
    srcp = jnp.concatenate(
        [edge_index[0], jnp.zeros((pad,), jnp.int32)]).reshape(
            NW, CHUNKS, CH)
    dstp = jnp.concatenate([edge_index[1], dummy]).reshape(NW, CHUNKS, CH)

    xw = x.reshape(NWR, 4 * D)
    b1aw = jnp.tile(b1a, 4).reshape(1, 128)
    b2aw = jnp.tile(b2a, 4).reshape(1, 128)
    b1bw = jnp.tile(b1b, 4).reshape(1, 128)
    b2bw = jnp.tile(b2b, 4).reshape(1, 128)

    y1w = pl.pallas_call(
        _mm1_body,
        out_shape=jax.ShapeDtypeStruct((NWIDE, 128), jnp.float32),
    )(xw, _bd(W1a))

    s1 = _build_edge_scatter()(y1w.reshape(N_PAD, H), srcp, dstp)

    z1w, y2w = pl.pallas_call(
        _mid_body,
        out_shape=(jax.ShapeDtypeStruct((NWR, 128), jnp.float32),
                   jax.ShapeDtypeStruct((NWIDE, 128), jnp.float32)),
    )(y1w, s1.reshape(NC * NWIDE, 128), b1aw, _bd(W2a), b2aw, _bd(W1b))

    s2 = _build_edge_scatter()(y2w.reshape(N_PAD, H), srcp, dstp)

    zw = pl.pallas_call(
        _out_body,
        out_shape=jax.ShapeDtypeStruct((NWR, 256), jnp.float32),
    )(z1w, y2w, s2.reshape(NC * NWIDE, 128), b1bw, _bd(W2b), b2bw)
    return zw.reshape(N, 2 * H)


# speedup vs baseline: 1.0737x; 1.0737x over previous
import functools

import jax
import jax.numpy as jnp
from jax import lax
from jax.experimental import pallas as pl
from jax.experimental.pallas import tpu as pltpu
from jax.experimental.pallas import tpu_sc as plsc

N = 10000
E = 320000
D = 128
H = 32

NC = 2
NS = 16
NW = NC * NS
CH = 128
K = 4
CHUNKS = 80
E_PAD = NW * CHUNKS * CH
NG = CHUNKS // K
ZR = 640
N_PAD = NS * ZR
ZB = 64
NWIDE = N_PAD // 4
NWR = N // 4


@functools.cache
def _build_edge_scatter():
    mesh = plsc.VectorSubcoreMesh(core_axis_name="c", subcore_axis_name="s")

    @functools.partial(
        pl.kernel,
        mesh=mesh,
        compiler_params=pltpu.CompilerParams(use_tc_tiling_on_sc=False),
        out_type=jax.ShapeDtypeStruct((NC * N_PAD, H), jnp.float32),
        scratch_types=[
            pltpu.VMEM((CHUNKS, CH), jnp.int32),
            pltpu.VMEM((CHUNKS, CH), jnp.int32),
            pltpu.VMEM((2 * K, CH, H), jnp.float32),
            pltpu.VMEM((ZB, H), jnp.float32),
            pltpu.VMEM((ZR, H), jnp.float32),
            pltpu.VMEM_SHARED((N_PAD, H), jnp.float32),
            pltpu.VMEM_SHARED((N_PAD, H), jnp.float32),
            pltpu.SemaphoreType.DMA,
            pltpu.SemaphoreType.DMA,
            pltpu.SemaphoreType.DMA,
            pltpu.SemaphoreType.DMA,
        ],
    )
    def _edge_scatter(y_hbm, src_hbm, dst_hbm, out_hbm,
                      src_v, dst_v, rows_v, zero_v, stage_v, acc_sh, y_sh,
                      gsem0, gsem1, ssem0, ssem1):
        c = lax.axis_index("c")
        s = lax.axis_index("s")
        wid = s * NC + c

        def _zrow(i, carry):
            zero_v[i, pl.ds(0, 16)] = jnp.zeros((16,), jnp.float32)
            zero_v[i, pl.ds(16, 16)] = jnp.zeros((16,), jnp.float32)
            return carry
        lax.fori_loop(0, ZB, _zrow, 0)
        base = s * ZR
        pro = []
        off = 0
        while off < ZR:
            k = min(ZB, ZR - off)
            pro.append(pltpu.async_copy(zero_v.at[pl.ds(0, k)],
                                        acc_sh.at[pl.ds(base + off, k)],
                                        gsem0))
            off += k
        yd = pltpu.async_copy(y_hbm.at[pl.ds(base, ZR)], stage_v, gsem1)
        pro.append(pltpu.async_copy(src_hbm.at[wid], src_v, ssem0))
        pro.append(pltpu.async_copy(dst_hbm.at[wid], dst_v, ssem1))
        yd.wait()
        pro.append(pltpu.async_copy(stage_v, y_sh.at[pl.ds(base, ZR)],
                                    gsem1))
        for d in pro:
            d.wait()
        plsc.subcore_barrier()

        gsems = (gsem0, gsem1)
        ssems = (ssem0, ssem1)
        gds = [None] * CHUNKS
        sds = [None] * CHUNKS

        def _fire_gathers(g):
            bank = g % 2
            for k in range(K):
                j = g * K + k
                gds[j] = pltpu.async_copy(
                    y_sh.at[src_v.at[j]], rows_v.at[bank * K + k],
                    gsems[bank])

        def _fire_scatters(g):
            bank = g % 2
            for k in range(K):
                j = g * K + k
                sds[j] = pltpu.async_copy(
                    rows_v.at[bank * K + k], acc_sh.at[dst_v.at[j]],
                    ssems[bank], add=True)

        _fire_gathers(0)
        for g in range(NG):
            if g + 1 < NG:
                if g >= 1:
                    for k in range(K):
                        sds[(g - 1) * K + k].wait()
                _fire_gathers(g + 1)
            for k in range(K):
                gds[g * K + k].wait()
            _fire_scatters(g)
        for g in (NG - 2, NG - 1):
            for k in range(K):
                sds[g * K + k].wait()
        plsc.subcore_barrier()

        pltpu.sync_copy(acc_sh.at[pl.ds(base, ZR)], stage_v)
        pltpu.sync_copy(stage_v, out_hbm.at[pl.ds(c * N_PAD + base, ZR)])

    return _edge_scatter


def _mm1_body(xw_ref, w_ref, o_ref):
    o_ref[0:NWR, :] = jnp.dot(xw_ref[...], w_ref[...],
                              preferred_element_type=jnp.float32)
    o_ref[NWR:NWIDE, :] = jnp.zeros((NWIDE - NWR, 128), jnp.float32)


def _mid_body(y1_ref, s1_ref, b1a_ref, w2a_ref, b2a_ref, w1b_ref,
              z1_ref, y2_ref):
    s1 = s1_ref[0:NWR, :] + s1_ref[NWIDE:NWIDE + NWR, :]
    t1 = jnp.maximum(y1_ref[0:NWR, :] + s1 + b1a_ref[...], 0.0)
    z1 = jnp.maximum(
        jnp.dot(t1, w2a_ref[...], preferred_element_type=jnp.float32)
        + b2a_ref[...], 0.0)
    z1_ref[...] = z1
    y2_ref[0:NWR, :] = jnp.dot(z1, w1b_ref[...],
                               preferred_element_type=jnp.float32)
    y2_ref[NWR:NWIDE, :] = jnp.zeros((NWIDE - NWR, 128), jnp.float32)


def _out_body(z1_ref, y2_ref, s2_ref, b1b_ref, w2b_ref, b2b_ref, z_ref):
    s2 = s2_ref[0:NWR, :] + s2_ref[NWIDE:NWIDE + NWR, :]
    t2 = jnp.maximum(y2_ref[0:NWR, :] + s2 + b1b_ref[...], 0.0)
    z2 = jnp.maximum(
        jnp.dot(t2, w2b_ref[...], preferred_element_type=jnp.float32)
        + b2b_ref[...], 0.0)
    z1 = z1_ref[...]
    parts = []
    for j in range(4):
        parts.append(z1[:, j * H:(j + 1) * H])
        parts.append(z2[:, j * H:(j + 1) * H])
    z_ref[...] = jnp.concatenate(parts, axis=1)


def _bd(w):
    return jnp.kron(jnp.eye(4, dtype=jnp.float32), w)


def kernel(x, edge_index, batch, W1a, b1a, W2a, b2a, W1b, b1b, W2b, b2b):
    per = E // NW
    padw = CHUNKS * CH - per
    src2 = edge_index[0].reshape(NW, per)
    dst2 = edge_index[1].reshape(NW, per)
    dummy = N + (jnp.arange(padw, dtype=jnp.int32) % (N_PAD - N))
    srcp = jnp.concatenate(
        [src2, jnp.zeros((NW, padw), jnp.int32)], axis=1).reshape(
            NW, CHUNKS, CH)
    dstp = jnp.concatenate(
        [dst2, jnp.broadcast_to(dummy, (NW, padw))], axis=1).reshape(
            NW, CHUNKS, CH)

    xw = x.reshape(NWR, 4 * D)
    b1aw = jnp.tile(b1a, 4).reshape(1, 128)
    b2aw = jnp.tile(b2a, 4).reshape(1, 128)
    b1bw = jnp.tile(b1b, 4).reshape(1, 128)
    b2bw = jnp.tile(b2b, 4).reshape(1, 128)

    y1w = pl.pallas_call(
        _mm1_body,
        out_shape=jax.ShapeDtypeStruct((NWIDE, 128), jnp.float32),
    )(xw, _bd(W1a))

    s1 = _build_edge_scatter()(y1w.reshape(N_PAD, H), srcp, dstp)

    z1w, y2w = pl.pallas_call(
        _mid_body,
        out_shape=(jax.ShapeDtypeStruct((NWR, 128), jnp.float32),
                   jax.ShapeDtypeStruct((NWIDE, 128), jnp.float32)),
    )(y1w, s1.reshape(NC * NWIDE, 128), b1aw, _bd(W2a), b2aw, _bd(W1b))

    s2 = _build_edge_scatter()(y2w.reshape(N_PAD, H), srcp, dstp)

    zw = pl.pallas_call(
        _out_body,
        out_shape=jax.ShapeDtypeStruct((NWR, 256), jnp.float32),
    )(z1w, y2w, s2.reshape(NC * NWIDE, 128), b1bw, _bd(W2b), b2bw)
    return zw.reshape(N, 2 * H)
